# baseline (speedup 1.0000x reference)
import functools

import jax
import jax.numpy as jnp
from jax import lax
from jax.experimental import pallas as pl
from jax.experimental.pallas import tpu as pltpu
from jax.experimental.pallas import tpu_sc as plsc

L = 16
NC, NS = 2, 16
NW = NC * NS

EMB = 64
KW = EMB // 2
NNEG = 200
SLOT = 208
NG = SLOT // L
CB = 4
ROWS = CB * SLOT
IDXBLK = 128


def _mf_kernel(nb, chunks,
               user_id, pos_items, neg_items, user_table, item_table,
               pos_out, neg_out,
               uid_v, pid_v, idx_d, rows_d, q_d, nbuf_d, pos_acc,
               sem_i, sem_r, sem_q, sem_o):
    wid = lax.axis_index("s") * NC + lax.axis_index("c")
    base = wid * nb
    lane = lax.broadcasted_iota(jnp.int32, (L,), 0)

    pltpu.sync_copy(user_id.at[pl.ds(base, nb)], uid_v)
    pltpu.sync_copy(pos_items.at[pl.ds(base, nb)], pid_v)

    def ids_descs(c, idx, p):
        cb0 = base + c * CB
        return [pltpu.make_async_copy(
            neg_items.at[pl.ds(pl.multiple_of((cb0 + b) * NNEG, 8), NNEG)],
            idx.at[pl.ds(b * SLOT, NNEG)], sem_i[p]) for b in range(CB)]

    def rows_descs(idx, rows):
        ds = []
        nfull, rem = ROWS // IDXBLK, ROWS % IDXBLK
        for j in range(nfull):
            ds.append(pltpu.make_async_copy(
                item_table.at[idx.at[pl.ds(j * IDXBLK, IDXBLK)]],
                rows.at[pl.ds(j * IDXBLK, IDXBLK)], sem_r))
        if rem:
            ds.append(pltpu.make_async_copy(
                item_table.at[idx.at[pl.ds(nfull * IDXBLK, rem)]],
                rows.at[pl.ds(nfull * IDXBLK, rem)], sem_r))
        return ds

    def q_desc(c, qb, p):
        qidx = plsc.load_gather(uid_v, [jnp.minimum(c * CB + lane, nb - 1)])
        return pltpu.make_async_copy(user_table.at[qidx], qb, sem_q[p])

    def scatter_pos(c, idx):
        pvals = plsc.load_gather(pid_v, [jnp.minimum(c * CB + lane, nb - 1)])
        plsc.store_scatter(idx, [lane * SLOT + NNEG], pvals, mask=lane < CB)

    def out_descs(c, nbuf, p):
        cb0 = base + c * CB
        return [pltpu.make_async_copy(
            nbuf.at[pl.ds(b * SLOT, NNEG)],
            neg_out.at[pl.ds(pl.multiple_of((cb0 + b) * NNEG, 8), NNEG)],
            sem_o[p]) for b in range(CB)]

    def compute(c, rows, qb, nbuf):
        for b in range(CB):
            qrow = jnp.full((L,), b, jnp.int32)
            row0 = b * SLOT

            def dbody(k, accs):
                col = (k + lane) & (KW - 1)
                qa = plsc.load_gather(qb, [qrow, col])
                qb2 = plsc.load_gather(qb, [qrow, col + KW])
                out = []
                for g in range(NG):
                    rw = plsc.load_gather(rows, [row0 + g * L + lane, col])
                    ra, rb = plsc.unpack(plsc.bitcast(rw, jnp.bfloat16),
                                         format=plsc.PackFormat.INTERLEAVED)
                    out.append(accs[g] + ra * qa + rb * qb2)
                return tuple(out)

            accs = lax.fori_loop(
                0, KW, dbody,
                tuple(jnp.zeros((L,), jnp.float32) for _ in range(NG)))
            for g in range(NG):
                nbuf[pl.ds(row0 + g * L, L)] = accs[g]
        pv = plsc.load_gather(
            nbuf, [jnp.minimum(lane * SLOT + NNEG, ROWS - 1)])
        plsc.store_scatter(pos_acc, [c * CB + lane], pv, mask=lane < CB)

    zeros_i = jnp.zeros((L,), jnp.int32)
    for p in range(2):
        for k in range(ROWS // L):
            idx_d[p][pl.ds(k * L, L)] = zeros_i

    for d in ids_descs(0, idx_d[0], 0):
        d.start()
    for d in ids_descs(0, idx_d[0], 0):
        d.wait()
    scatter_pos(0, idx_d[0])
    for d in rows_descs(idx_d[0], rows_d[0]):
        d.start()
    q_desc(0, q_d[0], 0).start()
    for d in ids_descs(1, idx_d[1], 1):
        d.start()

    def pair_body(cc, carry):
        for ph in range(2):
            c = 2 * cc + ph
            idx, rows, qb, nbuf = (
                idx_d[ph], rows_d[ph], q_d[ph], nbuf_d[ph])
            idxn, rowsn, qbn = (
                idx_d[1 - ph], rows_d[1 - ph], q_d[1 - ph])

            @pl.when(c + 1 < chunks)
            def _():
                for d in ids_descs(c + 1, idxn, 1 - ph):
                    d.wait()
                scatter_pos(c + 1, idxn)
                for d in rows_descs(idxn, rowsn):
                    d.start()
                q_desc(c + 1, qbn, 1 - ph).start()

            for d in rows_descs(idx, rows):
                d.wait()
            q_desc(c, qb, ph).wait()

            @pl.when(c + 2 < chunks)
            def _():
                for d in ids_descs(c + 2, idx, ph):
                    d.start()

            @pl.when(c >= 2)
            def _():
                for d in out_descs(c, nbuf, ph):
                    d.wait()

            compute(c, rows, qb, nbuf)
            for d in out_descs(c, nbuf, ph):
                d.start()
        return carry

    lax.fori_loop(0, chunks // 2, pair_body, 0)

    for d in out_descs(chunks - 2, nbuf_d[0], 0):
        d.wait()
    for d in out_descs(chunks - 1, nbuf_d[1], 1):
        d.wait()
    pltpu.sync_copy(pos_acc, pos_out.at[pl.ds(base, nb)])


def _pack_bf16(table):
    bits = jax.lax.bitcast_convert_type(table, jnp.int32)

    def rnd(x):
        return jax.lax.shift_right_logical(
            x + 0x7FFF + (jax.lax.shift_right_logical(x, 16) & 1), 16)

    return rnd(bits[:, :KW]) | (rnd(bits[:, KW:]) << 16)


def kernel(user_id, pos_items, neg_items, user_table, item_table):
    bsz = user_id.shape[0]
    n_neg = neg_items.shape[1]
    assert n_neg == NNEG and user_table.shape[1] == EMB
    assert bsz % (NW * 2 * CB) == 0
    nb = bsz // NW
    chunks = nb // CB

    mesh = plsc.VectorSubcoreMesh(
        core_axis_name="c", subcore_axis_name="s",
        num_cores=NC, num_subcores=NS)
    k = pl.kernel(
        functools.partial(_mf_kernel, nb, chunks),
        out_type=(
            jax.ShapeDtypeStruct((bsz,), jnp.float32),
            jax.ShapeDtypeStruct((bsz * n_neg,), jnp.float32),
        ),
        mesh=mesh,
        compiler_params=pltpu.CompilerParams(
            needs_layout_passes=False, use_tc_tiling_on_sc=False),
        scratch_types=[
            pltpu.VMEM((nb,), jnp.int32),
            pltpu.VMEM((nb,), jnp.int32),
            [pltpu.VMEM((ROWS,), jnp.int32)] * 2,
            [pltpu.VMEM((ROWS, KW), jnp.int32)] * 2,
            [pltpu.VMEM((L, EMB), jnp.float32)] * 2,
            [pltpu.VMEM((ROWS,), jnp.float32)] * 2,
            pltpu.VMEM((nb,), jnp.float32),
            [pltpu.SemaphoreType.DMA] * 2,
            pltpu.SemaphoreType.DMA,
            [pltpu.SemaphoreType.DMA] * 2,
            [pltpu.SemaphoreType.DMA] * 2,
        ],
        name="mf_scores_sc",
    )
    pos_score, neg_flat = k(
        user_id.astype(jnp.int32),
        pos_items.astype(jnp.int32),
        neg_items.reshape(-1).astype(jnp.int32),
        user_table,
        _pack_bf16(item_table),
    )
    return pos_score, neg_flat.reshape(bsz, n_neg)


# speedup vs baseline: 1.2449x; 1.2449x over previous
import functools

import jax
import jax.numpy as jnp
from jax import lax
from jax.experimental import pallas as pl
from jax.experimental.pallas import tpu as pltpu
from jax.experimental.pallas import tpu_sc as plsc

L = 16
NC, NS = 2, 16
NW = NC * NS

EMB = 64
KW = EMB // 2
NNEG = 200
SLOT = 208
NG = SLOT // L
CB = 4
ROWS = CB * SLOT
IDXBLK = 128


def _mf_kernel(nb, chunks,
               user_id, pos_items, neg_items, user_table, item_table,
               pos_out, neg_out,
               uid_v, pid_v, idx_d, rows_d, q_d, nbuf_d, pos_acc,
               sem_i, sem_r, sem_q, sem_o):
    wid = lax.axis_index("s") * NC + lax.axis_index("c")
    base = wid * nb
    lane = lax.broadcasted_iota(jnp.int32, (L,), 0)

    pltpu.sync_copy(user_id.at[pl.ds(base, nb)], uid_v)
    pltpu.sync_copy(pos_items.at[pl.ds(base, nb)], pid_v)

    def ids_descs(c, idx, p):
        cb0 = base + c * CB
        return [pltpu.make_async_copy(
            neg_items.at[pl.ds(pl.multiple_of((cb0 + b) * NNEG, 8), NNEG)],
            idx.at[pl.ds(b * SLOT, NNEG)], sem_i[p]) for b in range(CB)]

    def rows_descs(idx, rows):
        ds = []
        nfull, rem = ROWS // IDXBLK, ROWS % IDXBLK
        for j in range(nfull):
            ds.append(pltpu.make_async_copy(
                item_table.at[idx.at[pl.ds(j * IDXBLK, IDXBLK)]],
                rows.at[pl.ds(j * IDXBLK, IDXBLK)], sem_r))
        if rem:
            ds.append(pltpu.make_async_copy(
                item_table.at[idx.at[pl.ds(nfull * IDXBLK, rem)]],
                rows.at[pl.ds(nfull * IDXBLK, rem)], sem_r))
        return ds

    def q_desc(c, qb, p):
        qidx = plsc.load_gather(uid_v, [jnp.minimum(c * CB + lane, nb - 1)])
        return pltpu.make_async_copy(user_table.at[qidx], qb, sem_q[p])

    def scatter_pos(c, idx):
        pvals = plsc.load_gather(pid_v, [jnp.minimum(c * CB + lane, nb - 1)])
        plsc.store_scatter(idx, [lane * SLOT + NNEG], pvals, mask=lane < CB)

    def out_descs(c, nbuf, p):
        cb0 = base + c * CB
        return [pltpu.make_async_copy(
            nbuf.at[pl.ds(b * SLOT, NNEG)],
            neg_out.at[pl.ds(pl.multiple_of((cb0 + b) * NNEG, 8), NNEG)],
            sem_o[p]) for b in range(CB)]

    def compute(c, rows, qb, nbuf):
        for b in range(CB):
            qrow = jnp.full((L,), b, jnp.int32)
            row0 = b * SLOT

            def dbody(k, accs):
                col = (k + lane) & (KW - 1)
                qa = plsc.load_gather(qb, [qrow, col])
                qb2 = plsc.load_gather(qb, [qrow, col + KW])
                out = []
                for g in range(NG):
                    rw = plsc.load_gather(rows, [row0 + g * L + lane, col])
                    ra, rb = plsc.unpack(plsc.bitcast(rw, jnp.bfloat16),
                                         format=plsc.PackFormat.INTERLEAVED)
                    out.append(accs[g] + ra * qa + rb * qb2)
                return tuple(out)

            accs = lax.fori_loop(
                0, KW, dbody,
                tuple(jnp.zeros((L,), jnp.float32) for _ in range(NG)))
            for g in range(NG):
                nbuf[pl.ds(row0 + g * L, L)] = accs[g]
        pv = plsc.load_gather(
            nbuf, [jnp.minimum(lane * SLOT + NNEG, ROWS - 1)])
        plsc.store_scatter(pos_acc, [c * CB + lane], pv, mask=lane < CB)

    zeros_i = jnp.zeros((L,), jnp.int32)
    for p in range(2):
        for k in range(ROWS // L):
            idx_d[p][pl.ds(k * L, L)] = zeros_i

    for d in ids_descs(0, idx_d[0], 0):
        d.start()
    for d in ids_descs(0, idx_d[0], 0):
        d.wait()
    scatter_pos(0, idx_d[0])
    for d in rows_descs(idx_d[0], rows_d[0]):
        d.start()
    q_desc(0, q_d[0], 0).start()
    for d in ids_descs(1, idx_d[1], 1):
        d.start()

    def pair_body(cc, carry):
        for ph in range(2):
            c = 2 * cc + ph
            idx, rows, qb, nbuf = (
                idx_d[ph], rows_d[ph], q_d[ph], nbuf_d[ph])
            idxn, rowsn, qbn = (
                idx_d[1 - ph], rows_d[1 - ph], q_d[1 - ph])

            @pl.when(c + 1 < chunks)
            def _():
                for d in ids_descs(c + 1, idxn, 1 - ph):
                    d.wait()
                scatter_pos(c + 1, idxn)
                for d in rows_descs(idxn, rowsn):
                    d.start()
                q_desc(c + 1, qbn, 1 - ph).start()

            for d in rows_descs(idx, rows):
                d.wait()
            q_desc(c, qb, ph).wait()

            @pl.when(c + 2 < chunks)
            def _():
                for d in ids_descs(c + 2, idx, ph):
                    d.start()

            @pl.when(c >= 2)
            def _():
                for d in out_descs(c, nbuf, ph):
                    d.wait()

            compute(c, rows, qb, nbuf)
            for d in out_descs(c, nbuf, ph):
                d.start()
        return carry

    lax.fori_loop(0, chunks // 2, pair_body, 0)

    for d in out_descs(chunks - 2, nbuf_d[0], 0):
        d.wait()
    for d in out_descs(chunks - 1, nbuf_d[1], 1):
        d.wait()
    pltpu.sync_copy(pos_acc, pos_out.at[pl.ds(base, nb)])


PKR = 512


def _rne16(x):
    return jax.lax.shift_right_logical(
        x + 0x7FFF + (jax.lax.shift_right_logical(x, 16) & 1), 16)


def _pack_kernel(pt, nblk, tail, table, out, fin_d, fout_d, sem_in, sem_out):
    wid = lax.axis_index("s") * NC + lax.axis_index("c")
    base = wid * pt

    def in_desc(b, p):
        return pltpu.make_async_copy(
            table.at[pl.ds(base + b * PKR, PKR)], fin_d[p], sem_in[p])

    def out_desc(b, p):
        return pltpu.make_async_copy(
            fout_d[p], out.at[pl.ds(base + b * PKR, PKR)], sem_out[p])

    def pack_rows(fin, fout, nr):
        def body(r, carry):
            for h in (0, L):
                lo = jax.lax.bitcast_convert_type(
                    fin[r, pl.ds(h, L)], jnp.int32)
                hi = jax.lax.bitcast_convert_type(
                    fin[r, pl.ds(h + KW, L)], jnp.int32)
                fout[r, pl.ds(h, L)] = _rne16(lo) | (_rne16(hi) << 16)
            return carry
        lax.fori_loop(0, nr, body, 0)

    in_desc(0, 0).start()
    for b in range(nblk):
        p = b & 1
        in_desc(b, p).wait()
        if b + 1 < nblk:
            in_desc(b + 1, 1 - p).start()
        if b >= 2:
            out_desc(b, p).wait()
        pack_rows(fin_d[p], fout_d[p], PKR)
        out_desc(b, p).start()
    if nblk >= 2:
        out_desc(nblk - 2, nblk & 1).wait()
    if nblk >= 1:
        out_desc(nblk - 1, 1 - (nblk & 1)).wait()

    if tail:
        t0 = base + nblk * PKR
        pltpu.sync_copy(table.at[pl.ds(t0, tail)],
                        fin_d[0].at[pl.ds(0, tail)])
        pack_rows(fin_d[0], fout_d[0], tail)
        pltpu.sync_copy(fout_d[0].at[pl.ds(0, tail)],
                        out.at[pl.ds(t0, tail)])

    @pl.when(wid == 0)
    def _():
        last = NW * pt
        pltpu.sync_copy(table.at[pl.ds(last, 1)], fin_d[0].at[pl.ds(0, 1)])
        pack_rows(fin_d[0], fout_d[0], 1)
        pltpu.sync_copy(fout_d[0].at[pl.ds(0, 1)], out.at[pl.ds(last, 1)])


def _pack_bf16_sc(table, mesh, params):
    ni = table.shape[0]
    assert (ni - 1) % NW == 0
    pt = (ni - 1) // NW
    nblk, tail = pt // PKR, pt % PKR
    k = pl.kernel(
        functools.partial(_pack_kernel, pt, nblk, tail),
        out_type=jax.ShapeDtypeStruct((ni, KW), jnp.int32),
        mesh=mesh,
        compiler_params=params,
        scratch_types=[
            [pltpu.VMEM((PKR, EMB), jnp.float32)] * 2,
            [pltpu.VMEM((PKR, KW), jnp.int32)] * 2,
            [pltpu.SemaphoreType.DMA] * 2,
            [pltpu.SemaphoreType.DMA] * 2,
        ],
        name="mf_pack_sc",
    )
    return k(table)


def kernel(user_id, pos_items, neg_items, user_table, item_table):
    bsz = user_id.shape[0]
    n_neg = neg_items.shape[1]
    assert n_neg == NNEG and user_table.shape[1] == EMB
    assert bsz % (NW * 2 * CB) == 0
    nb = bsz // NW
    chunks = nb // CB

    mesh = plsc.VectorSubcoreMesh(
        core_axis_name="c", subcore_axis_name="s",
        num_cores=NC, num_subcores=NS)
    params = pltpu.CompilerParams(
        needs_layout_passes=False, use_tc_tiling_on_sc=False)
    k = pl.kernel(
        functools.partial(_mf_kernel, nb, chunks),
        out_type=(
            jax.ShapeDtypeStruct((bsz,), jnp.float32),
            jax.ShapeDtypeStruct((bsz * n_neg,), jnp.float32),
        ),
        mesh=mesh,
        compiler_params=params,
        scratch_types=[
            pltpu.VMEM((nb,), jnp.int32),
            pltpu.VMEM((nb,), jnp.int32),
            [pltpu.VMEM((ROWS,), jnp.int32)] * 2,
            [pltpu.VMEM((ROWS, KW), jnp.int32)] * 2,
            [pltpu.VMEM((L, EMB), jnp.float32)] * 2,
            [pltpu.VMEM((ROWS,), jnp.float32)] * 2,
            pltpu.VMEM((nb,), jnp.float32),
            [pltpu.SemaphoreType.DMA] * 2,
            pltpu.SemaphoreType.DMA,
            [pltpu.SemaphoreType.DMA] * 2,
            [pltpu.SemaphoreType.DMA] * 2,
        ],
        name="mf_scores_sc",
    )
    pos_score, neg_flat = k(
        user_id.astype(jnp.int32),
        pos_items.astype(jnp.int32),
        neg_items.reshape(-1).astype(jnp.int32),
        user_table,
        _pack_bf16_sc(item_table, mesh, params),
    )
    return pos_score, neg_flat.reshape(bsz, n_neg)


# speedup vs baseline: 1.2477x; 1.0023x over previous
import functools

import jax
import jax.numpy as jnp
from jax import lax
from jax.experimental import pallas as pl
from jax.experimental.pallas import tpu as pltpu
from jax.experimental.pallas import tpu_sc as plsc

L = 16
NC, NS = 2, 16
NW = NC * NS

EMB = 64
KW = EMB // 2
NNEG = 200
SLOT = 208
NG = SLOT // L
CB = 4
ROWS = CB * SLOT
IDXBLK = 128


def _mf_kernel(nb, chunks,
               user_id, pos_items, neg_items, user_table, item_table,
               pos_out, neg_out,
               uid_v, pid_v, idx_d, rows_d, q_d, nbuf_d, pos_acc,
               sem_i, sem_r, sem_q, sem_o):
    wid = lax.axis_index("s") * NC + lax.axis_index("c")
    base = wid * nb
    lane = lax.broadcasted_iota(jnp.int32, (L,), 0)

    pltpu.sync_copy(user_id.at[pl.ds(base, nb)], uid_v)
    pltpu.sync_copy(pos_items.at[pl.ds(base, nb)], pid_v)

    def ids_descs(c, idx, p):
        cb0 = base + c * CB
        return [pltpu.make_async_copy(
            neg_items.at[pl.ds(pl.multiple_of((cb0 + b) * NNEG, 8), NNEG)],
            idx.at[pl.ds(b * SLOT, NNEG)], sem_i[p]) for b in range(CB)]

    def rows_descs(idx, rows):
        ds = []
        nfull, rem = ROWS // IDXBLK, ROWS % IDXBLK
        for j in range(nfull):
            ds.append(pltpu.make_async_copy(
                item_table.at[idx.at[pl.ds(j * IDXBLK, IDXBLK)]],
                rows.at[pl.ds(j * IDXBLK, IDXBLK)], sem_r))
        if rem:
            ds.append(pltpu.make_async_copy(
                item_table.at[idx.at[pl.ds(nfull * IDXBLK, rem)]],
                rows.at[pl.ds(nfull * IDXBLK, rem)], sem_r))
        return ds

    def q_desc(c, qb, p):
        qidx = plsc.load_gather(uid_v, [jnp.minimum(c * CB + lane, nb - 1)])
        return pltpu.make_async_copy(user_table.at[qidx], qb, sem_q[p])

    def scatter_pos(c, idx):
        pvals = plsc.load_gather(pid_v, [jnp.minimum(c * CB + lane, nb - 1)])
        plsc.store_scatter(idx, [lane * SLOT + NNEG], pvals, mask=lane < CB)

    def out_descs(c, nbuf, p):
        cb0 = base + c * CB
        return [pltpu.make_async_copy(
            nbuf.at[pl.ds(b * SLOT, NNEG)],
            neg_out.at[pl.ds(pl.multiple_of((cb0 + b) * NNEG, 8), NNEG)],
            sem_o[p]) for b in range(CB)]

    def compute(c, rows, qb, nbuf):
        for b in range(CB):
            qrow = jnp.full((L,), b, jnp.int32)
            row0 = b * SLOT

            def dbody(k, accs):
                col = (k + lane) & (KW - 1)
                qa = plsc.load_gather(qb, [qrow, col])
                qb2 = plsc.load_gather(qb, [qrow, col + KW])
                out = []
                for g in range(NG):
                    rw = plsc.load_gather(rows, [row0 + g * L + lane, col])
                    ra, rb = plsc.unpack(plsc.bitcast(rw, jnp.bfloat16),
                                         format=plsc.PackFormat.INTERLEAVED)
                    out.append(accs[g] + ra * qa + rb * qb2)
                return tuple(out)

            accs = lax.fori_loop(
                0, KW, dbody,
                tuple(jnp.zeros((L,), jnp.float32) for _ in range(NG)))
            for g in range(NG):
                nbuf[pl.ds(row0 + g * L, L)] = accs[g]
        pv = plsc.load_gather(
            nbuf, [jnp.minimum(lane * SLOT + NNEG, ROWS - 1)])
        plsc.store_scatter(pos_acc, [c * CB + lane], pv, mask=lane < CB)

    zeros_i = jnp.zeros((L,), jnp.int32)
    for p in range(2):
        for k in range(ROWS // L):
            idx_d[p][pl.ds(k * L, L)] = zeros_i

    for d in ids_descs(0, idx_d[0], 0):
        d.start()
    for d in ids_descs(0, idx_d[0], 0):
        d.wait()
    scatter_pos(0, idx_d[0])
    for d in rows_descs(idx_d[0], rows_d[0]):
        d.start()
    q_desc(0, q_d[0], 0).start()
    for d in ids_descs(1, idx_d[1], 1):
        d.start()

    def pair_body(cc, carry):
        for ph in range(2):
            c = 2 * cc + ph
            idx, rows, qb, nbuf = (
                idx_d[ph], rows_d[ph], q_d[ph], nbuf_d[ph])
            idxn, rowsn, qbn = (
                idx_d[1 - ph], rows_d[1 - ph], q_d[1 - ph])

            @pl.when(c + 1 < chunks)
            def _():
                for d in ids_descs(c + 1, idxn, 1 - ph):
                    d.wait()
                scatter_pos(c + 1, idxn)
                for d in rows_descs(idxn, rowsn):
                    d.start()
                q_desc(c + 1, qbn, 1 - ph).start()

            for d in rows_descs(idx, rows):
                d.wait()
            q_desc(c, qb, ph).wait()

            @pl.when(c + 2 < chunks)
            def _():
                for d in ids_descs(c + 2, idx, ph):
                    d.start()

            @pl.when(c >= 2)
            def _():
                for d in out_descs(c, nbuf, ph):
                    d.wait()

            compute(c, rows, qb, nbuf)
            for d in out_descs(c, nbuf, ph):
                d.start()
        return carry

    lax.fori_loop(0, chunks // 2, pair_body, 0)

    for d in out_descs(chunks - 2, nbuf_d[0], 0):
        d.wait()
    for d in out_descs(chunks - 1, nbuf_d[1], 1):
        d.wait()
    pltpu.sync_copy(pos_acc, pos_out.at[pl.ds(base, nb)])


PKR = 512


def _rne16(x):
    return jax.lax.shift_right_logical(
        x + 0x7FFF + (jax.lax.shift_right_logical(x, 16) & 1), 16)


def _pack_kernel(pt, nblk, tail, table, out, fin_d, fout_d, sem_in, sem_out):
    wid = lax.axis_index("s") * NC + lax.axis_index("c")
    base = wid * pt

    def in_desc(b, p):
        return pltpu.make_async_copy(
            table.at[pl.ds(base + b * PKR, PKR)], fin_d[p], sem_in[p])

    def out_desc(b, p):
        return pltpu.make_async_copy(
            fout_d[p], out.at[pl.ds(base + b * PKR, PKR)], sem_out[p])

    def pack_rows(fin, fout, nr):
        def body(i, carry):
            r0 = i * 2
            for r in (r0, jnp.minimum(r0 + 1, nr - 1)):
                for h in (0, L):
                    lo = jax.lax.bitcast_convert_type(
                        fin[r, pl.ds(h, L)], jnp.int32)
                    hi = jax.lax.bitcast_convert_type(
                        fin[r, pl.ds(h + KW, L)], jnp.int32)
                    fout[r, pl.ds(h, L)] = _rne16(lo) | (_rne16(hi) << 16)
            return carry
        lax.fori_loop(0, (nr + 1) // 2, body, 0)

    in_desc(0, 0).start()
    for b in range(nblk):
        p = b & 1
        in_desc(b, p).wait()
        if b + 1 < nblk:
            in_desc(b + 1, 1 - p).start()
        if b >= 2:
            out_desc(b, p).wait()
        pack_rows(fin_d[p], fout_d[p], PKR)
        out_desc(b, p).start()
    if nblk >= 2:
        out_desc(nblk - 2, nblk & 1).wait()
    if nblk >= 1:
        out_desc(nblk - 1, 1 - (nblk & 1)).wait()

    if tail:
        t0 = base + nblk * PKR
        pltpu.sync_copy(table.at[pl.ds(t0, tail)],
                        fin_d[0].at[pl.ds(0, tail)])
        pack_rows(fin_d[0], fout_d[0], tail)
        pltpu.sync_copy(fout_d[0].at[pl.ds(0, tail)],
                        out.at[pl.ds(t0, tail)])

    @pl.when(wid == 0)
    def _():
        last = NW * pt
        pltpu.sync_copy(table.at[pl.ds(last, 1)], fin_d[0].at[pl.ds(0, 1)])
        pack_rows(fin_d[0], fout_d[0], 1)
        pltpu.sync_copy(fout_d[0].at[pl.ds(0, 1)], out.at[pl.ds(last, 1)])


def _pack_bf16_sc(table, mesh, params):
    ni = table.shape[0]
    assert (ni - 1) % NW == 0
    pt = (ni - 1) // NW
    nblk, tail = pt // PKR, pt % PKR
    k = pl.kernel(
        functools.partial(_pack_kernel, pt, nblk, tail),
        out_type=jax.ShapeDtypeStruct((ni, KW), jnp.int32),
        mesh=mesh,
        compiler_params=params,
        scratch_types=[
            [pltpu.VMEM((PKR, EMB), jnp.float32)] * 2,
            [pltpu.VMEM((PKR, KW), jnp.int32)] * 2,
            [pltpu.SemaphoreType.DMA] * 2,
            [pltpu.SemaphoreType.DMA] * 2,
        ],
        name="mf_pack_sc",
    )
    return k(table)


def kernel(user_id, pos_items, neg_items, user_table, item_table):
    bsz = user_id.shape[0]
    n_neg = neg_items.shape[1]
    assert n_neg == NNEG and user_table.shape[1] == EMB
    assert bsz % (NW * 2 * CB) == 0
    nb = bsz // NW
    chunks = nb // CB

    mesh = plsc.VectorSubcoreMesh(
        core_axis_name="c", subcore_axis_name="s",
        num_cores=NC, num_subcores=NS)
    params = pltpu.CompilerParams(
        needs_layout_passes=False, use_tc_tiling_on_sc=False)
    k = pl.kernel(
        functools.partial(_mf_kernel, nb, chunks),
        out_type=(
            jax.ShapeDtypeStruct((bsz,), jnp.float32),
            jax.ShapeDtypeStruct((bsz * n_neg,), jnp.float32),
        ),
        mesh=mesh,
        compiler_params=params,
        scratch_types=[
            pltpu.VMEM((nb,), jnp.int32),
            pltpu.VMEM((nb,), jnp.int32),
            [pltpu.VMEM((ROWS,), jnp.int32)] * 2,
            [pltpu.VMEM((ROWS, KW), jnp.int32)] * 2,
            [pltpu.VMEM((L, EMB), jnp.float32)] * 2,
            [pltpu.VMEM((ROWS,), jnp.float32)] * 2,
            pltpu.VMEM((nb,), jnp.float32),
            [pltpu.SemaphoreType.DMA] * 2,
            pltpu.SemaphoreType.DMA,
            [pltpu.SemaphoreType.DMA] * 2,
            [pltpu.SemaphoreType.DMA] * 2,
        ],
        name="mf_scores_sc",
    )
    pos_score, neg_flat = k(
        user_id.astype(jnp.int32),
        pos_items.astype(jnp.int32),
        neg_items.reshape(-1).astype(jnp.int32),
        user_table,
        _pack_bf16_sc(item_table, mesh, params),
    )
    return pos_score, neg_flat.reshape(bsz, n_neg)
